# baseline (speedup 1.0000x reference)
import jax
import jax.numpy as jnp
from jax import lax
from jax.experimental import pallas as pl
from jax.experimental.pallas import tpu as pltpu
from jax.experimental.pallas import tpu_sc as plsc

NC, NS = 2, 16
NW = NC * NS
B = 16384
D = 128
BPW = B // NW
C = 5


R = 128
NK = BPW // R


def _gather_body(user_t, game_t, user_idx, game_idx, uout, gout,
                 idx_u, idx_g, ub0, ub1, gb0, gb1,
                 su0, su1, sg0, sg1, wu0, wu1, wg0, wg1):
    wid = lax.axis_index("s") * NC + lax.axis_index("c")
    base = wid * BPW
    pltpu.sync_copy(user_idx.at[pl.ds(base, BPW)], idx_u)
    pltpu.sync_copy(game_idx.at[pl.ds(base, BPW)], idx_g)

    ubufs, gbufs = (ub0, ub1), (gb0, gb1)
    gsems = ((su0, sg0), (su1, sg1))
    wsems = ((wu0, wg0), (wu1, wg1))

    def start_gather(k):
        s = k % 2
        return (pltpu.async_copy(user_t.at[idx_u.at[pl.ds(k * R, R)]],
                                 ubufs[s], gsems[s][0]),
                pltpu.async_copy(game_t.at[idx_g.at[pl.ds(k * R, R)]],
                                 gbufs[s], gsems[s][1]))

    def start_wb(k):
        s = k % 2
        dst = pl.ds(base + k * R, R)
        return (pltpu.async_copy(ubufs[s], uout.at[dst], wsems[s][0]),
                pltpu.async_copy(gbufs[s], gout.at[dst], wsems[s][1]))

    g = start_gather(0)
    wb_prev = None
    for k in range(NK):
        g[0].wait()
        g[1].wait()
        wb = start_wb(k)
        if k + 1 < NK:
            if wb_prev is not None:
                wb_prev[0].wait()
                wb_prev[1].wait()
            g = start_gather(k + 1)
        wb_prev, wb = wb, None
    wb_prev[0].wait()
    wb_prev[1].wait()


_sc_gather = pl.kernel(
    _gather_body,
    out_type=(jax.ShapeDtypeStruct((B, D), jnp.float32),
              jax.ShapeDtypeStruct((B, D), jnp.float32)),
    mesh=plsc.VectorSubcoreMesh(core_axis_name="c", subcore_axis_name="s"),
    scratch_types=[
        pltpu.VMEM((BPW,), jnp.int32),
        pltpu.VMEM((BPW,), jnp.int32),
        pltpu.VMEM((R, D), jnp.float32),
        pltpu.VMEM((R, D), jnp.float32),
        pltpu.VMEM((R, D), jnp.float32),
        pltpu.VMEM((R, D), jnp.float32),
    ] + [pltpu.SemaphoreType.DMA] * 8,
)

_DN = (((1,), (1,)), ((), ()))


def _matmul_body(u_ref, g_ref, w_ref, b_ref, o_ref):
    w = w_ref[...]
    acc = lax.dot_general(u_ref[...], w[:, :D], _DN,
                          preferred_element_type=jnp.float32)
    acc += lax.dot_general(g_ref[...], w[:, D:], _DN,
                           preferred_element_type=jnp.float32)
    o_ref[...] = acc + b_ref[...]


def _tc_matmul(uemb, gemb, fc_w, bias):
    bm = 4096
    grid = (B // bm,)
    return pl.pallas_call(
        _matmul_body,
        grid=grid,
        in_specs=[
            pl.BlockSpec((bm, D), lambda i: (i, 0)),
            pl.BlockSpec((bm, D), lambda i: (i, 0)),
            pl.BlockSpec((C, 2 * D), lambda i: (0, 0)),
            pl.BlockSpec((1, C), lambda i: (0, 0)),
        ],
        out_specs=pl.BlockSpec((bm, C), lambda i: (i, 0)),
        out_shape=jax.ShapeDtypeStruct((B, C), jnp.float32),
    )(uemb, gemb, fc_w, bias)


def kernel(user, game, user_table, game_table, fc_w, fc_b):
    uemb, gemb = _sc_gather(user_table, game_table, user, game)
    return _tc_matmul(uemb, gemb, fc_w, fc_b.reshape(1, C))


# speedup vs baseline: 1.0199x; 1.0199x over previous
import jax
import jax.numpy as jnp
from jax import lax
from jax.experimental import pallas as pl
from jax.experimental.pallas import tpu as pltpu
from jax.experimental.pallas import tpu_sc as plsc

NC, NS = 2, 16
NW = NC * NS
B = 16384
D = 128
BPW = B // NW
C = 5


def _gather_body(user_t, game_t, user_idx, game_idx, uout, gout,
                 idx_v, rows_v, sem):
    wid = lax.axis_index("s") * NC + lax.axis_index("c")
    base = wid * BPW
    pltpu.sync_copy(user_idx.at[pl.ds(base, BPW)], idx_v)
    pltpu.async_copy(user_t.at[idx_v], rows_v, sem).wait()
    pltpu.sync_copy(rows_v, uout.at[pl.ds(base, BPW)])
    pltpu.sync_copy(game_idx.at[pl.ds(base, BPW)], idx_v)
    pltpu.async_copy(game_t.at[idx_v], rows_v, sem).wait()
    pltpu.sync_copy(rows_v, gout.at[pl.ds(base, BPW)])


_sc_gather = pl.kernel(
    _gather_body,
    out_type=(jax.ShapeDtypeStruct((B, D), jnp.float32),
              jax.ShapeDtypeStruct((B, D), jnp.float32)),
    mesh=plsc.VectorSubcoreMesh(core_axis_name="c", subcore_axis_name="s"),
    scratch_types=[
        pltpu.VMEM((BPW,), jnp.int32),
        pltpu.VMEM((BPW, D), jnp.float32),
        pltpu.SemaphoreType.DMA,
    ],
)

_DN = (((1,), (1,)), ((), ()))


def _matmul_body(u_ref, g_ref, w_ref, b_ref, o_ref):
    w = w_ref[...]
    acc = lax.dot_general(u_ref[...], w[:, :D], _DN,
                          preferred_element_type=jnp.float32)
    acc += lax.dot_general(g_ref[...], w[:, D:], _DN,
                           preferred_element_type=jnp.float32)
    o_ref[...] = acc + b_ref[...]


def _tc_matmul(uemb, gemb, fc_w, bias):
    bm = 4096
    grid = (B // bm,)
    return pl.pallas_call(
        _matmul_body,
        grid=grid,
        in_specs=[
            pl.BlockSpec((bm, D), lambda i: (i, 0)),
            pl.BlockSpec((bm, D), lambda i: (i, 0)),
            pl.BlockSpec((C, 2 * D), lambda i: (0, 0)),
            pl.BlockSpec((1, C), lambda i: (0, 0)),
        ],
        out_specs=pl.BlockSpec((bm, C), lambda i: (i, 0)),
        out_shape=jax.ShapeDtypeStruct((B, C), jnp.float32),
    )(uemb, gemb, fc_w, bias)


def kernel(user, game, user_table, game_table, fc_w, fc_b):
    uemb, gemb = _sc_gather(user_table, game_table, user, game)
    return _tc_matmul(uemb, gemb, fc_w, fc_b.reshape(1, C))
